# Initial kernel scaffold; baseline (speedup 1.0000x reference)
import jax
import jax.numpy as jnp
from jax.experimental import pallas as pl


def kernel(inputs, table):
    raise NotImplementedError("write your pallas kernel here")


# speedup vs baseline: 2.5198x; 2.5198x over previous
import functools
import math

import jax
import jax.numpy as jnp
from jax import lax
from jax.experimental import pallas as pl
from jax.experimental.pallas import tpu as pltpu
from jax.experimental.pallas import tpu_sc as plsc

D_MODEL = 128
LANES = 16
NUM_CORES = 2
NUM_SUBCORES = 16
NUM_WORKERS = NUM_CORES * NUM_SUBCORES
CHUNK = 128
SCALE = math.sqrt(D_MODEL)


@functools.lru_cache(maxsize=None)
def _make_sc_gather(n_chunks: int):
    mesh = plsc.VectorSubcoreMesh(
        core_axis_name="c", subcore_axis_name="s", num_cores=NUM_CORES
    )
    n_rows = NUM_WORKERS * n_chunks * CHUNK

    @functools.partial(
        pl.kernel,
        mesh=mesh,
        out_type=jax.ShapeDtypeStruct((n_rows, D_MODEL), jnp.float32),
        scratch_types=[
            pltpu.VMEM((n_chunks, CHUNK), jnp.int32),
            pltpu.VMEM((CHUNK, D_MODEL), jnp.float32),
            pltpu.SemaphoreType.DMA,
        ],
    )
    def sc_gather(table_hbm, idx_hbm, out_hbm, idx_v, buf, sem):
        wid = lax.axis_index("s") * NUM_CORES + lax.axis_index("c")
        row0 = wid * n_chunks
        pltpu.sync_copy(idx_hbm.at[wid], idx_v)

        def chunk_body(c, carry):
            pltpu.async_copy(table_hbm.at[idx_v.at[c]], buf, sem).wait()

            def scale_row(i, carry2):
                for j in range(D_MODEL // LANES):
                    sl = pl.ds(j * LANES, LANES)
                    buf[i, sl] = buf[i, sl] * SCALE
                return carry2

            lax.fori_loop(0, CHUNK, scale_row, 0)
            pltpu.sync_copy(buf, out_hbm.at[pl.ds((row0 + c) * CHUNK, CHUNK)])
            return carry

        lax.fori_loop(0, n_chunks, chunk_body, 0)

    return sc_gather


def kernel(inputs, table):
    bsz, seq = inputs.shape
    idx = inputs.reshape(-1).astype(jnp.int32)
    n_total = idx.shape[0]
    per_worker = n_total // NUM_WORKERS
    n_chunks = per_worker // CHUNK
    idx2 = idx.reshape(NUM_WORKERS, n_chunks, CHUNK)
    out = _make_sc_gather(n_chunks)(table.astype(jnp.float32), idx2)
    return out.reshape(bsz, seq, D_MODEL)


# speedup vs baseline: 3.0025x; 1.1916x over previous
import functools
import math

import jax
import jax.numpy as jnp
from jax import lax
from jax.experimental import pallas as pl
from jax.experimental.pallas import tpu as pltpu
from jax.experimental.pallas import tpu_sc as plsc

D_MODEL = 128
LANES = 16
NUM_CORES = 2
NUM_SUBCORES = 16
NUM_WORKERS = NUM_CORES * NUM_SUBCORES
CHUNK = 128
SCALE = math.sqrt(D_MODEL)


@functools.lru_cache(maxsize=None)
def _make_sc_gather(n_chunks: int):
    mesh = plsc.VectorSubcoreMesh(
        core_axis_name="c", subcore_axis_name="s", num_cores=NUM_CORES
    )
    n_rows = NUM_WORKERS * n_chunks * CHUNK

    assert n_chunks >= 4 and n_chunks % 2 == 0

    @functools.partial(
        pl.kernel,
        mesh=mesh,
        out_type=jax.ShapeDtypeStruct((n_rows, D_MODEL), jnp.float32),
        scratch_types=[
            pltpu.VMEM((n_chunks, CHUNK), jnp.int32),
            pltpu.VMEM((CHUNK, D_MODEL), jnp.float32),
            pltpu.VMEM((CHUNK, D_MODEL), jnp.float32),
            pltpu.SemaphoreType.DMA,
            pltpu.SemaphoreType.DMA,
            pltpu.SemaphoreType.DMA,
            pltpu.SemaphoreType.DMA,
        ],
    )
    def sc_gather(
        table_hbm, idx_hbm, out_hbm, idx_v, buf0, buf1, gsem0, gsem1, ssem0, ssem1
    ):
        wid = lax.axis_index("s") * NUM_CORES + lax.axis_index("c")
        row0 = wid * n_chunks
        pltpu.sync_copy(idx_hbm.at[wid], idx_v)
        bufs = (buf0, buf1)
        gsems = (gsem0, gsem1)
        ssems = (ssem0, ssem1)

        def gather_start(c, b):
            pltpu.async_copy(table_hbm.at[idx_v.at[c]], bufs[b], gsems[b])

        def gather_wait(c, b):
            pltpu.make_async_copy(table_hbm.at[idx_v.at[c]], bufs[b], gsems[b]).wait()

        def scale(b):
            buf = bufs[b]

            def scale_rows(r, carry):
                for k in range(4):
                    for j in range(D_MODEL // LANES):
                        sl = pl.ds(j * LANES, LANES)
                        buf[r * 4 + k, sl] = buf[r * 4 + k, sl] * SCALE
                return carry

            lax.fori_loop(0, CHUNK // 4, scale_rows, 0)

        def scatter_start(c, b):
            pltpu.async_copy(
                bufs[b], out_hbm.at[pl.ds((row0 + c) * CHUNK, CHUNK)], ssems[b]
            )

        def scatter_wait(c, b):
            pltpu.make_async_copy(
                bufs[b], out_hbm.at[pl.ds((row0 + c) * CHUNK, CHUNK)], ssems[b]
            ).wait()

        gather_start(0, 0)
        gather_start(1, 1)
        gather_wait(0, 0)
        scale(0)
        scatter_start(0, 0)

        def pair(g, carry):
            c = 2 * g + 1
            scatter_wait(c - 1, 0)
            gather_start(c + 1, 0)
            gather_wait(c, 1)
            scale(1)
            scatter_start(c, 1)

            scatter_wait(c, 1)
            gather_start(c + 2, 1)
            gather_wait(c + 1, 0)
            scale(0)
            scatter_start(c + 1, 0)
            return carry

        lax.fori_loop(0, (n_chunks - 2) // 2, pair, 0)

        c_last = n_chunks - 1
        gather_wait(c_last, 1)
        scale(1)
        scatter_start(c_last, 1)
        scatter_wait(c_last - 1, 0)
        scatter_wait(c_last, 1)

    return sc_gather


def kernel(inputs, table):
    bsz, seq = inputs.shape
    idx = inputs.reshape(-1).astype(jnp.int32)
    n_total = idx.shape[0]
    per_worker = n_total // NUM_WORKERS
    n_chunks = per_worker // CHUNK
    idx2 = idx.reshape(NUM_WORKERS, n_chunks, CHUNK)
    out = _make_sc_gather(n_chunks)(table.astype(jnp.float32), idx2)
    return out.reshape(bsz, seq, D_MODEL)


# speedup vs baseline: 3.0550x; 1.0175x over previous
import functools
import math

import jax
import jax.numpy as jnp
from jax import lax
from jax.experimental import pallas as pl
from jax.experimental.pallas import tpu as pltpu
from jax.experimental.pallas import tpu_sc as plsc

D_MODEL = 128
LANES = 16
NUM_CORES = 2
NUM_SUBCORES = 16
NUM_WORKERS = NUM_CORES * NUM_SUBCORES
CHUNK = 128
SCALE = math.sqrt(D_MODEL)


@functools.lru_cache(maxsize=None)
def _make_sc_gather(n_chunks: int):
    mesh = plsc.VectorSubcoreMesh(
        core_axis_name="c", subcore_axis_name="s", num_cores=NUM_CORES
    )
    n_rows = NUM_WORKERS * n_chunks * CHUNK

    NBUF = 4
    assert n_chunks >= 8 and n_chunks % 2 == 0

    @functools.partial(
        pl.kernel,
        mesh=mesh,
        out_type=jax.ShapeDtypeStruct((n_rows, D_MODEL), jnp.float32),
        scratch_types=[
            pltpu.VMEM((n_chunks, CHUNK), jnp.int32),
            [pltpu.VMEM((CHUNK, D_MODEL), jnp.float32) for _ in range(NBUF)],
            [pltpu.SemaphoreType.DMA for _ in range(NBUF)],
            [pltpu.SemaphoreType.DMA for _ in range(NBUF)],
        ],
    )
    def sc_gather(table_hbm, idx_hbm, out_hbm, idx_v, bufs, gsems, ssems):
        wid = lax.axis_index("s") * NUM_CORES + lax.axis_index("c")
        row0 = wid * n_chunks
        pltpu.sync_copy(idx_hbm.at[wid], idx_v)

        def gather_start(c, b):
            pltpu.async_copy(table_hbm.at[idx_v.at[c]], bufs[b], gsems[b])

        def gather_wait(c, b):
            pltpu.make_async_copy(table_hbm.at[idx_v.at[c]], bufs[b], gsems[b]).wait()

        def scale(b):
            buf = bufs[b]

            def scale_rows(r, carry):
                for k in range(4):
                    for j in range(D_MODEL // LANES):
                        sl = pl.ds(j * LANES, LANES)
                        buf[r * 4 + k, sl] = buf[r * 4 + k, sl] * SCALE
                return carry

            lax.fori_loop(0, CHUNK // 4, scale_rows, 0)

        def scatter_start(c, b):
            pltpu.async_copy(
                bufs[b], out_hbm.at[pl.ds((row0 + c) * CHUNK, CHUNK)], ssems[b]
            )

        def scatter_wait(c, b):
            pltpu.make_async_copy(
                bufs[b], out_hbm.at[pl.ds((row0 + c) * CHUNK, CHUNK)], ssems[b]
            ).wait()

        def step(c):
            b = c % NBUF
            b2 = (c + 2) % NBUF
            if c >= 2 and c + 2 < n_chunks:
                scatter_wait(c - 2, b2)
            if c + 2 < n_chunks:
                gather_start(c + 2, b2)
            gather_wait(c, b)
            scale(b)
            scatter_start(c, b)

        gather_start(0, 0)
        gather_start(1, 1)
        step(0)
        step(1)

        head = 2
        tail = 4 + ((n_chunks - 2) % 4)
        n_quads = (n_chunks - head - tail) // 4

        def quad(q, carry):
            c0 = head + 4 * q
            for k in range(4):
                c = c0 + k
                b = (head + k) % NBUF
                b2 = (head + k + 2) % NBUF
                scatter_wait(c - 2, b2)
                gather_start(c + 2, b2)
                gather_wait(c, b)
                scale(b)
                scatter_start(c, b)
            return carry

        lax.fori_loop(0, n_quads, quad, 0)

        for c in range(head + 4 * n_quads, n_chunks):
            step(c)

        for c in range(n_chunks - NBUF, n_chunks):
            scatter_wait(c, c % NBUF)

    return sc_gather


def kernel(inputs, table):
    bsz, seq = inputs.shape
    idx = inputs.reshape(-1).astype(jnp.int32)
    n_total = idx.shape[0]
    per_worker = n_total // NUM_WORKERS
    n_chunks = per_worker // CHUNK
    idx2 = idx.reshape(NUM_WORKERS, n_chunks, CHUNK)
    out = _make_sc_gather(n_chunks)(table.astype(jnp.float32), idx2)
    return out.reshape(bsz, seq, D_MODEL)


# speedup vs baseline: 5.4122x; 1.7716x over previous
import functools
import math

import jax
import jax.numpy as jnp
from jax import lax
from jax.experimental import pallas as pl
from jax.experimental.pallas import tpu as pltpu
from jax.experimental.pallas import tpu_sc as plsc

D_MODEL = 128
LANES = 16
NUM_CORES = 2
NUM_SUBCORES = 16
NUM_WORKERS = NUM_CORES * NUM_SUBCORES
BPB = 2
NBUF = 4
SCALE = math.sqrt(D_MODEL)


@functools.lru_cache(maxsize=None)
def _make_sc_gather(n_batches: int, seq: int):
    mesh = plsc.VectorSubcoreMesh(
        core_axis_name="c", subcore_axis_name="s", num_cores=NUM_CORES
    )
    bpw = n_batches // NUM_WORKERS
    n_steps = bpw // BPB
    assert n_steps >= 8 and n_steps % 2 == 0

    @functools.partial(
        pl.kernel,
        mesh=mesh,
        out_type=jax.ShapeDtypeStruct((n_batches, seq, D_MODEL), jnp.float32),
        scratch_types=[
            pltpu.VMEM((bpw, seq), jnp.int32),
            [pltpu.VMEM((BPB, seq, D_MODEL), jnp.float32) for _ in range(NBUF)],
            [pltpu.SemaphoreType.DMA for _ in range(NBUF)],
            [pltpu.SemaphoreType.DMA for _ in range(NBUF)],
        ],
    )
    def sc_gather(table_hbm, idx_hbm, out_hbm, idx_v, bufs, gsems, ssems):
        wid = lax.axis_index("s") * NUM_CORES + lax.axis_index("c")
        batch0 = wid * bpw
        pltpu.sync_copy(idx_hbm.at[pl.ds(batch0, bpw)], idx_v)

        def gather_start(c, b):
            for g in range(BPB):
                pltpu.async_copy(
                    table_hbm.at[idx_v.at[c * BPB + g]], bufs[b].at[g], gsems[b]
                )

        def gather_wait(c, b):
            for g in range(BPB):
                pltpu.make_async_copy(
                    table_hbm.at[idx_v.at[c * BPB + g]], bufs[b].at[g], gsems[b]
                ).wait()

        def scale(b):
            buf = bufs[b]

            def scale_rows(r, carry):
                for g in range(BPB):
                    for j in range(D_MODEL // LANES):
                        sl = pl.ds(j * LANES, LANES)
                        buf[g, r, sl] = buf[g, r, sl] * SCALE
                return carry

            lax.fori_loop(0, seq, scale_rows, 0)

        def scatter_start(c, b):
            pltpu.async_copy(
                bufs[b], out_hbm.at[pl.ds(batch0 + c * BPB, BPB)], ssems[b]
            )

        def scatter_wait(c, b):
            pltpu.make_async_copy(
                bufs[b], out_hbm.at[pl.ds(batch0 + c * BPB, BPB)], ssems[b]
            ).wait()

        def step(c):
            b = c % NBUF
            b2 = (c + 2) % NBUF
            if c >= 2 and c + 2 < n_steps:
                scatter_wait(c - 2, b2)
            if c + 2 < n_steps:
                gather_start(c + 2, b2)
            gather_wait(c, b)
            scale(b)
            scatter_start(c, b)

        gather_start(0, 0)
        gather_start(1, 1)
        step(0)
        step(1)

        head = 2
        tail = 4 + ((n_steps - 2) % 4)
        n_quads = (n_steps - head - tail) // 4

        def quad(q, carry):
            c0 = head + 4 * q
            for k in range(4):
                c = c0 + k
                b = (head + k) % NBUF
                b2 = (head + k + 2) % NBUF
                scatter_wait(c - 2, b2)
                gather_start(c + 2, b2)
                gather_wait(c, b)
                scale(b)
                scatter_start(c, b)
            return carry

        lax.fori_loop(0, n_quads, quad, 0)

        for c in range(head + 4 * n_quads, n_steps):
            step(c)

        for c in range(n_steps - NBUF, n_steps):
            scatter_wait(c, c % NBUF)

    return sc_gather


def kernel(inputs, table):
    bsz, seq = inputs.shape
    idx = inputs.astype(jnp.int32)
    return _make_sc_gather(bsz, seq)(table.astype(jnp.float32), idx)
